# baseline (speedup 1.0000x reference)
import functools

import jax
import jax.numpy as jnp
from jax import lax
from jax.experimental import pallas as pl
from jax.experimental.pallas import tpu as pltpu
from jax.experimental.pallas import tpu_sc as plsc

NUM_CLASSES = 1000000
OUT_FEATURES = 64
BATCH = 16384
PAIR = 2 * OUT_FEATURES
BLK = 16384
NBLK = 32
SPLIT = BLK * NBLK
MAXBLK = (NUM_CLASSES - 1) // BLK

_INFO = plsc.get_sparse_core_info()
_NC, _NS, _L = _INFO.num_cores, _INFO.num_subcores, _INFO.num_lanes
_NW = _NC * _NS
_BPW = BATCH // _NW
_CHUNK = 128
_NCHUNK = _BPW // _CHUNK

_mesh = plsc.VectorSubcoreMesh(core_axis_name="c", subcore_axis_name="s")


def _pack_body(xl_ref, xr_ref, o_ref):
    xl = xl_ref[...]
    xr = xr_ref[...]
    ident = jnp.eye(64, dtype=jnp.float32)
    dims = (((0,), (0,)), ((), ()))
    yl = lax.dot_general(xl, ident, dims, preferred_element_type=jnp.float32)
    yr = lax.dot_general(xr, ident, dims, preferred_element_type=jnp.float32)
    o_ref[...] = jnp.concatenate([yl, yr], axis=1)


def _pack(tab_t):
    return pl.pallas_call(
        _pack_body,
        grid=(NBLK,),
        in_specs=[
            pl.BlockSpec((64, BLK), lambda j: (0, j)),
            pl.BlockSpec((64, BLK), lambda j: (0, jnp.minimum(j + NBLK, MAXBLK))),
        ],
        out_specs=pl.BlockSpec((BLK, PAIR), lambda j: (j, 0)),
        out_shape=jax.ShapeDtypeStruct((SPLIT, PAIR), jnp.float32),
    )(tab_t, tab_t)


@functools.partial(
    pl.kernel,
    mesh=_mesh,
    out_type=jax.ShapeDtypeStruct((BATCH, OUT_FEATURES), jnp.float32),
    scratch_types=[
        pltpu.VMEM((_BPW,), jnp.int32),
        pltpu.VMEM((_BPW,), jnp.int32),
        pltpu.VMEM((2, _CHUNK, PAIR), jnp.float32),
        pltpu.VMEM((_BPW, OUT_FEATURES), jnp.float32),
        pltpu.SemaphoreType.DMA,
        pltpu.SemaphoreType.DMA,
    ],
    compiler_params=pltpu.CompilerParams(
        use_tc_tiling_on_sc=True, needs_layout_passes=False),
)
def _gather_kernel(pid_hbm, hoff_hbm, table_hbm, out_hbm,
                   pid_v, hoff_v, pairs_v, out_v, sem0, sem1):
    wid = lax.axis_index("s") * _NC + lax.axis_index("c")
    base = wid * _BPW
    pltpu.sync_copy(pid_hbm.at[pl.ds(base, _BPW)], pid_v)
    pltpu.sync_copy(hoff_hbm.at[pl.ds(base, _BPW)], hoff_v)

    copies = [None, None]
    sems = [sem0, sem1]

    def start(c):
        copies[c % 2] = pltpu.async_copy(
            table_hbm.at[pid_v.at[pl.ds(c * _CHUNK, _CHUNK)]],
            pairs_v.at[c % 2],
            sems[c % 2],
        )

    start(0)
    start(1)
    for c in range(_NCHUNK):
        copies[c % 2].wait()
        buf = pairs_v.at[c % 2]

        def select(g, carry, c=c, buf=buf):
            off = c * _CHUNK + g * _L
            jj = off + lax.iota(jnp.int32, _L)
            jl = g * _L + lax.iota(jnp.int32, _L)
            half = hoff_v[pl.ds(off, _L)]
            for col in range(OUT_FEATURES):
                cc = jnp.full((_L,), col, jnp.int32)
                vals = plsc.load_gather(buf, [jl, half + cc])
                plsc.store_scatter(out_v, [jj, cc], vals)
            return carry

        lax.fori_loop(0, _CHUNK // _L, select, 0)
        if c + 2 < _NCHUNK:
            start(c + 2)

    pltpu.sync_copy(out_v, out_hbm.at[pl.ds(base, _BPW)])


def kernel(class_ids, table):
    idx = class_ids.reshape(BATCH).astype(jnp.int32)
    pid = jnp.where(idx >= SPLIT, idx - SPLIT, idx)
    hoff = jnp.where(idx >= SPLIT, OUT_FEATURES, 0).astype(jnp.int32)
    packed = _pack(table.T)
    out = _gather_kernel(pid, hoff, packed)
    return out.reshape(BATCH, 1, OUT_FEATURES)


# speedup vs baseline: 1.0960x; 1.0960x over previous
import functools

import jax
import jax.numpy as jnp
from jax import lax
from jax.experimental import pallas as pl
from jax.experimental.pallas import tpu as pltpu
from jax.experimental.pallas import tpu_sc as plsc

NUM_CLASSES = 1000000
OUT_FEATURES = 64
BATCH = 16384
PAIR = 2 * OUT_FEATURES
BLK = 16384
NBLK = 32
SPLIT = BLK * NBLK
MAXBLK = (NUM_CLASSES - 1) // BLK

_INFO = plsc.get_sparse_core_info()
_NC, _NS = _INFO.num_cores, _INFO.num_subcores
_NW = _NC * _NS
_BPW = BATCH // _NW
_CHUNK = 128
_NCHUNK = _BPW // _CHUNK

_mesh = plsc.VectorSubcoreMesh(core_axis_name="c", subcore_axis_name="s")


def _pack_body(xl_ref, xr_ref, o_ref):
    xl = xl_ref[...]
    xr = xr_ref[...]
    ident = jnp.eye(64, dtype=jnp.float32)
    dims = (((0,), (0,)), ((), ()))
    yl = lax.dot_general(xl, ident, dims, preferred_element_type=jnp.float32)
    yr = lax.dot_general(xr, ident, dims, preferred_element_type=jnp.float32)
    o_ref[...] = jnp.concatenate([yl, yr], axis=1)


def _pack(tab_t):
    return pl.pallas_call(
        _pack_body,
        grid=(NBLK,),
        in_specs=[
            pl.BlockSpec((64, BLK), lambda j: (0, j)),
            pl.BlockSpec((64, BLK), lambda j: (0, jnp.minimum(j + NBLK, MAXBLK))),
        ],
        out_specs=pl.BlockSpec((BLK, PAIR), lambda j: (j, 0)),
        out_shape=jax.ShapeDtypeStruct((SPLIT, PAIR), jnp.float32),
    )(tab_t, tab_t)


@functools.partial(
    pl.kernel,
    mesh=_mesh,
    out_type=jax.ShapeDtypeStruct((BATCH, PAIR), jnp.float32),
    scratch_types=[
        pltpu.VMEM((_BPW,), jnp.int32),
        pltpu.VMEM((_BPW, PAIR), jnp.float32),
        pltpu.SemaphoreType.DMA,
    ],
    compiler_params=pltpu.CompilerParams(use_tc_tiling_on_sc=True),
)
def _gather_kernel(pid_hbm, table_hbm, out_hbm, pid_v, rows_v, sem):
    wid = lax.axis_index("s") * _NC + lax.axis_index("c")
    base = wid * _BPW
    pltpu.sync_copy(pid_hbm.at[pl.ds(base, _BPW)], pid_v)
    copies = []
    for c in range(_NCHUNK):
        copies.append(
            pltpu.async_copy(
                table_hbm.at[pid_v.at[pl.ds(c * _CHUNK, _CHUNK)]],
                rows_v.at[pl.ds(c * _CHUNK, _CHUNK)],
                sem,
            )
        )
    for c in copies:
        c.wait()
    pltpu.sync_copy(rows_v, out_hbm.at[pl.ds(base, _BPW)])


def _select_body(sel_ref, pairs_ref, o_ref):
    x = pairs_ref[...]
    s = sel_ref[...]
    o_ref[...] = jnp.where(s > 0, x[:, OUT_FEATURES:], x[:, :OUT_FEATURES])


def _select(sel, pairs):
    return pl.pallas_call(
        _select_body,
        grid=(BATCH // 512,),
        in_specs=[
            pl.BlockSpec((512, 1), lambda j: (j, 0)),
            pl.BlockSpec((512, PAIR), lambda j: (j, 0)),
        ],
        out_specs=pl.BlockSpec((512, OUT_FEATURES), lambda j: (j, 0)),
        out_shape=jax.ShapeDtypeStruct((BATCH, OUT_FEATURES), jnp.float32),
    )(sel, pairs)


def kernel(class_ids, table):
    idx = class_ids.reshape(BATCH).astype(jnp.int32)
    pid = jnp.where(idx >= SPLIT, idx - SPLIT, idx)
    sel = (idx >= SPLIT).reshape(BATCH, 1)
    packed = _pack(table.T)
    pairs = _gather_kernel(pid, packed)
    out = jnp.where(sel, pairs[:, OUT_FEATURES:], pairs[:, :OUT_FEATURES])
    return out.reshape(BATCH, 1, OUT_FEATURES)
